# baseline (speedup 1.0000x reference)
import functools

import jax
import jax.numpy as jnp
from jax import lax
from jax.experimental import pallas as pl
from jax.experimental.pallas import tpu as pltpu
from jax.experimental.pallas import tpu_sc as plsc

_B = 4096
_F = 26
_V = 1000000
_VROW = 1000000
_NW = 32
_BPW = _B // _NW
_L = 16
_NCH = _BPW // _L


def _make_kernel():
    mesh = plsc.VectorSubcoreMesh(core_axis_name="c", subcore_axis_name="s")

    @functools.partial(
        pl.kernel,
        mesh=mesh,
        out_type=jax.ShapeDtypeStruct((_B,), jnp.float32),
        compiler_params=pltpu.CompilerParams(
            needs_layout_passes=False, use_tc_tiling_on_sc=False),
        scratch_types=[
            pltpu.VMEM((_BPW * _F,), jnp.int32),
            pltpu.VMEM((_F * _BPW, 1), jnp.float32),
            pltpu.VMEM((_BPW,), jnp.float32),
            pltpu.SemaphoreType.DMA,
        ],
    )
    def k(x_hbm, tab_hbm, out_hbm, xv, valsv, outv, sem):
        wid = lax.axis_index("s") * 2 + lax.axis_index("c")
        base = wid * _BPW

        tab_rows = tab_hbm.at[0]

        pltpu.sync_copy(x_hbm.at[pl.ds(base * _F, _BPW * _F)], xv)

        lane = lax.iota(jnp.int32, 16)
        zero16 = lane * 0
        copies = []
        for f in range(_F):
            for c in range(_NCH):
                pos = lane * _F + (16 * c * _F + f)
                xg = plsc.load_gather(xv, [pos])
                flat = xg + f * _VROW
                copies.append(pltpu.async_copy(
                    tab_rows.at[flat],
                    valsv.at[pl.ds(f * _BPW + 16 * c, 16), :], sem))
        for cp in copies:
            cp.wait()

        for c in range(_NCH):
            acc = plsc.load_gather(valsv, [lane + 16 * c, zero16])
            for f in range(1, _F):
                acc = acc + plsc.load_gather(
                    valsv, [lane + f * _BPW + 16 * c, zero16])
            outv[pl.ds(16 * c, 16)] = acc

        pltpu.sync_copy(outv, out_hbm.at[pl.ds(base, _BPW)])

    return k


_sc_kernel = _make_kernel()


def kernel(X, tables):
    x_flat = X.reshape(_B * _F).astype(jnp.int32)
    out = _sc_kernel(x_flat, tables)
    return out.reshape(_B, 1)


# speedup vs baseline: 31.6827x; 31.6827x over previous
import functools

import jax
import jax.numpy as jnp
from jax import lax
from jax.experimental import pallas as pl
from jax.experimental.pallas import tpu as pltpu
from jax.experimental.pallas import tpu_sc as plsc

_B = 4096
_F = 26
_V = 1000000
_NW = 32
_BPW = _B // _NW
_L = 16
_NCH = _BPW // _L


def _make_kernel():
    mesh = plsc.VectorSubcoreMesh(core_axis_name="c", subcore_axis_name="s")

    @functools.partial(
        pl.kernel,
        mesh=mesh,
        out_type=jax.ShapeDtypeStruct((_B,), jnp.float32),
        compiler_params=pltpu.CompilerParams(needs_layout_passes=False),
        scratch_types=[
            pltpu.VMEM((_BPW * _F,), jnp.int32),
            pltpu.VMEM((_F * _BPW,), jnp.float32),
            pltpu.VMEM((_BPW,), jnp.float32),
            pltpu.SemaphoreType.DMA,
        ],
    )
    def k(x_hbm, tab_hbm, out_hbm, xv, valsv, outv, sem):
        wid = lax.axis_index("s") * 2 + lax.axis_index("c")
        base = wid * _BPW

        pltpu.sync_copy(x_hbm.at[pl.ds(base * _F, _BPW * _F)], xv)

        lane = lax.iota(jnp.int32, 16)
        copies = []
        for f in range(_F):
            for c in range(_NCH):
                pos = lane * _F + (16 * c * _F + f)
                xg = plsc.load_gather(xv, [pos])
                flat = xg + f * _V
                copies.append(pltpu.async_copy(
                    tab_hbm.at[flat],
                    valsv.at[pl.ds(f * _BPW + 16 * c, 16)], sem))
        for cp in copies:
            cp.wait()

        for c in range(_NCH):
            acc = valsv[pl.ds(16 * c, 16)]
            for f in range(1, _F):
                acc = acc + valsv[pl.ds(f * _BPW + 16 * c, 16)]
            outv[pl.ds(16 * c, 16)] = acc

        pltpu.sync_copy(outv, out_hbm.at[pl.ds(base, _BPW)])

    return k


_sc_kernel = _make_kernel()


def kernel(X, tables):
    x_flat = X.reshape(_B * _F).astype(jnp.int32)
    tab_flat = jnp.concatenate([tables[f, :, 0] for f in range(_F)])
    out = _sc_kernel(x_flat, tab_flat)
    return out.reshape(_B, 1)
